# baseline (speedup 1.0000x reference)
import jax
import jax.numpy as jnp
from jax import lax
from jax.experimental import pallas as pl
from jax.experimental.pallas import tpu as pltpu
from jax.experimental.pallas import tpu_sc as plsc

NC = 2
NS = 16
NW = NC * NS
L = 16
DBLK = 8


def _tec_body(data_hbm, ts_hbm, pe_hbm, out_hbm, pe_c, ts_v,
              data_v0, data_v1, sem_in0, sem_in1, sem_ts0, sem_ts1,
              sem_out0, sem_out1, sem_pe):
    hist, d, batch = data_hbm.shape
    v = pe_hbm.shape[0] // d
    ngrp = d // DBLK
    nq = NW // ngrp
    bq = batch // nq
    wid = lax.axis_index("s") * NC + lax.axis_index("c")
    g = wid // nq
    q = wid % nq
    dbase = g * DBLK
    bbase = q * bq
    sem_in = (sem_in0, sem_in1)
    sem_ts = (sem_ts0, sem_ts1)
    sem_out = (sem_out0, sem_out1)
    data_v = (data_v0, data_v1)

    def in_desc(h, b):
        return pltpu.make_async_copy(
            data_hbm.at[h, pl.ds(dbase, DBLK), pl.ds(bbase, bq)],
            data_v[b], sem_in[b])

    def ts_desc(h, b):
        return pltpu.make_async_copy(
            ts_hbm.at[pl.ds(h * batch + bbase, bq)],
            ts_v.at[pl.ds(b * bq, bq)], sem_ts[b])

    def fire_in(h, b):
        in_desc(h, b).start()
        ts_desc(h, b).start()

    def out_desc(h, b):
        return pltpu.make_async_copy(
            data_v[b],
            out_hbm.at[h, pl.ds(dbase, DBLK), pl.ds(bbase, bq)], sem_out[b])

    def fire_work(h, b):
        ts_desc(h, b).wait()
        in_desc(h, b).wait()

        @plsc.parallel_loop(0, bq // L, 1, unroll=4)
        def add_body(i):
            s16 = pl.ds(i * L, L)
            t = ts_v[pl.ds(b * bq + i * L, L)]
            iv = ((t.astype(jnp.float32) + 0.5) * 0.01).astype(jnp.int32)
            for dl in range(DBLK):
                rows = plsc.load_gather(pe_c, [iv + (dl * v)])
                data_v[b][dl, s16] = data_v[b][dl, s16] + rows
        out_desc(h, b).start()

    for dl in range(DBLK):
        pltpu.async_copy(
            pe_hbm.at[pl.ds((dbase + dl) * v, v)],
            pe_c.at[pl.ds(dl * v, v)], sem_pe).wait()

    fire_in(0, 0)
    fire_in(1, 1)
    fire_work(0, 0)

    def pair_body(p, _):
        h0 = p * 2
        out_desc(h0, 0).wait()
        fire_in(h0 + 2, 0)
        fire_work(h0 + 1, 1)
        out_desc(h0 + 1, 1).wait()
        fire_in(h0 + 3, 1)
        fire_work(h0 + 2, 0)
        return 0

    lax.fori_loop(0, hist // 2 - 1, pair_body, 0)

    hl = hist - 2
    out_desc(hl, 0).wait()
    fire_work(hl + 1, 1)
    out_desc(hl + 1, 1).wait()


@jax.jit
def kernel(data, time_stamps, pe):
    b, h, d = data.shape
    v = pe.shape[0]
    data_t = jnp.transpose(data, (1, 2, 0))
    ts_flat = jnp.transpose(time_stamps, (1, 0)).reshape(h * b)
    pe_flat = jnp.transpose(pe, (1, 0)).reshape(d * v)

    mesh = plsc.VectorSubcoreMesh(
        core_axis_name="c", subcore_axis_name="s", num_cores=NC,
        num_subcores=NS)
    bq = b // (NW // (d // DBLK))
    out_t = pl.kernel(
        _tec_body,
        out_type=jax.ShapeDtypeStruct((h, d, b), jnp.float32),
        mesh=mesh,
        scratch_types=[
            pltpu.VMEM((DBLK * v,), jnp.float32),
            pltpu.VMEM((2 * bq,), jnp.int32),
            pltpu.VMEM((DBLK, bq), jnp.float32),
            pltpu.VMEM((DBLK, bq), jnp.float32),
            pltpu.SemaphoreType.DMA,
            pltpu.SemaphoreType.DMA,
            pltpu.SemaphoreType.DMA,
            pltpu.SemaphoreType.DMA,
            pltpu.SemaphoreType.DMA,
            pltpu.SemaphoreType.DMA,
            pltpu.SemaphoreType.DMA,
        ],
        compiler_params=pltpu.CompilerParams(needs_layout_passes=False),
    )(data_t, ts_flat, pe_flat)
    return jnp.transpose(out_t, (2, 0, 1))
